# baseline (speedup 1.0000x reference)
import jax
import jax.numpy as jnp
from jax.experimental import pallas as pl
from jax.experimental.pallas import tpu as pltpu

_NUM_CLASSES = 81
_K = 12544
_NA = 512
_NB = 512
_N = _NA + _NB
_H = 1024
_RT = 256
_EPS = 1e-3

_PIECES = (
    ((0, 0, 256), (1, 0, 256), (2, 0, 256), (3, 0, 128)),
    ((3, 128, 256), (4, 0, 256), (5, 0, 256), (6, 0, 256)),
)


def _dot(a, b):
    return jnp.dot(a, b, preferred_element_type=jnp.float32)


def _dot_rt(a, bt):
    return jax.lax.dot_general(a, bt, (((1,), (1,)), ((), ())),
                               preferred_element_type=jnp.float32)


def _head_kernel(a_ref, b_ref, w1_ref, g1_ref, be1_ref,
                 w2_ref, g2_ref, be2_ref,
                 lw_ref, lb_ref, dw_ref, db_ref,
                 logits_ref, probs_ref, deltas_ref, acc_ref, x2_ref):
    k = pl.program_id(0)

    def partial(x3, wblk, pieces):
        ps, r = None, 0
        for wi, c0, c1 in pieces:
            d = _dot(x3[wi][:, c0:c1], wblk[r:r + (c1 - c0), :])
            r += c1 - c0
            ps = d if ps is None else ps + d
        return ps

    def partials(pieces):
        wblk = w1_ref[0]
        return partial(a_ref[0], wblk, pieces), partial(b_ref[0], wblk, pieces)

    @pl.when(k == 0)
    def _():
        pa, pb = partials(_PIECES[0])
        acc_ref[0:_NA, :] = pa
        acc_ref[_NA:_N, :] = pb

    @pl.when((k > 0) & (k % 2 == 0))
    def _():
        pa, pb = partials(_PIECES[0])
        acc_ref[0:_NA, :] += pa
        acc_ref[_NA:_N, :] += pb

    @pl.when((k % 2 == 1) & (k < 13))
    def _():
        pa, pb = partials(_PIECES[1])
        acc_ref[0:_NA, :] += pa
        acc_ref[_NA:_N, :] += pb

    @pl.when(k == 13)
    def _():
        zero = jnp.zeros((1, _H), jnp.float32)
        pa, pb = partials(_PIECES[1])

        ta = acc_ref[0:_NA, :] + pa
        tb = acc_ref[_NA:_N, :] + pb
        acc_ref[0:_NA, :] = ta
        acc_ref[_NA:_N, :] = tb
        s1 = jnp.sum(ta, axis=0, keepdims=True) + jnp.sum(tb, axis=0, keepdims=True)
        ss1 = (jnp.sum(ta * ta, axis=0, keepdims=True) +
               jnp.sum(tb * tb, axis=0, keepdims=True))
        mean1 = s1 * (1.0 / _N)
        var1 = ss1 * (1.0 / _N) - mean1 * mean1
        scale1 = g1_ref[...] * jax.lax.rsqrt(var1 + _EPS)
        shift1 = be1_ref[...] - mean1 * scale1

        def body1(i, carry):
            s, ss = carry
            r = i * _RT
            xt = jnp.maximum(acc_ref[pl.ds(r, _RT), :] * scale1 + shift1, 0.0)
            x2t = _dot(xt, w2_ref[...])
            x2_ref[pl.ds(r, _RT), :] = x2t
            return (s + jnp.sum(x2t, axis=0, keepdims=True),
                    ss + jnp.sum(x2t * x2t, axis=0, keepdims=True))

        s2, ss2 = jax.lax.fori_loop(0, _N // _RT, body1, (zero, zero))
        mean2 = s2 * (1.0 / _N)
        var2 = ss2 * (1.0 / _N) - mean2 * mean2
        scale2 = g2_ref[...] * jax.lax.rsqrt(var2 + _EPS)
        shift2 = be2_ref[...] - mean2 * scale2

        def body2(i, carry):
            r = i * _RT
            xt = jnp.maximum(x2_ref[pl.ds(r, _RT), :] * scale2 + shift2, 0.0)
            lg = _dot_rt(lw_ref[...], xt) + lb_ref[...]
            logits_ref[:, pl.ds(r, _RT)] = lg
            m = jnp.max(lg, axis=0, keepdims=True)
            e = jnp.exp(lg - m)
            probs_ref[:, pl.ds(r, _RT)] = e / jnp.sum(e, axis=0, keepdims=True)
            deltas_ref[:, pl.ds(r, _RT)] = _dot_rt(dw_ref[...], xt) + db_ref[...]
            return carry

        jax.lax.fori_loop(0, _N // _RT, body2, 0)


def kernel(pooled_rois_a, pooled_rois_b, conv1_w, conv1_b, bn1_gamma, bn1_beta,
           conv2_w, conv2_b, bn2_gamma, bn2_beta, logits_w, logits_b,
           delta_w, delta_b):
    del conv1_b, conv2_b
    a4 = jnp.transpose(pooled_rois_a, (1, 2, 0, 3))
    b4 = jnp.transpose(pooled_rois_b, (1, 2, 0, 3))
    w3 = conv1_w.reshape(14, _K // 14, _H)
    lwt = logits_w.T
    dwt = delta_w.T
    row = lambda v: v.reshape(1, -1)
    col = lambda v: v.reshape(-1, 1)
    full = lambda shape: pl.BlockSpec(shape, lambda k: tuple(0 for _ in shape))

    logits_t, probs_t, deltas_t = pl.pallas_call(
        _head_kernel,
        grid=(14,),
        in_specs=[
            pl.BlockSpec((1, 7, _NA, 256), lambda k: (k // 2, 0, 0, 0)),
            pl.BlockSpec((1, 7, _NB, 256), lambda k: (k // 2, 0, 0, 0)),
            pl.BlockSpec((1, _K // 14, _H), lambda k: (k, 0, 0)),
            full((1, _H)), full((1, _H)),
            full((_H, _H)), full((1, _H)), full((1, _H)),
            full((_NUM_CLASSES, _H)), full((_NUM_CLASSES, 1)),
            full((4 * _NUM_CLASSES, _H)), full((4 * _NUM_CLASSES, 1)),
        ],
        out_specs=[
            full((_NUM_CLASSES, _N)),
            full((_NUM_CLASSES, _N)),
            full((4 * _NUM_CLASSES, _N)),
        ],
        out_shape=[
            jax.ShapeDtypeStruct((_NUM_CLASSES, _N), jnp.float32),
            jax.ShapeDtypeStruct((_NUM_CLASSES, _N), jnp.float32),
            jax.ShapeDtypeStruct((4 * _NUM_CLASSES, _N), jnp.float32),
        ],
        scratch_shapes=[
            pltpu.VMEM((_N, _H), jnp.float32),
            pltpu.VMEM((_N, _H), jnp.float32),
        ],
        compiler_params=pltpu.CompilerParams(
            dimension_semantics=("arbitrary",)),
    )(a4, b4, w3,
      row(bn1_gamma), row(bn1_beta),
      conv2_w, row(bn2_gamma), row(bn2_beta),
      lwt, col(logits_b), dwt, col(delta_b))

    logits = logits_t.T
    probs = probs_t.T
    deltas = deltas_t.reshape(_NUM_CLASSES, 4, _N).transpose(2, 0, 1)
    return (logits, probs, deltas)


# speedup vs baseline: 1.1423x; 1.1423x over previous
import jax
import jax.numpy as jnp
from jax.experimental import pallas as pl
from jax.experimental.pallas import tpu as pltpu

_NUM_CLASSES = 81
_K = 12544
_NA = 512
_NB = 512
_N = _NA + _NB
_H = 1024
_RT = 256
_EPS = 1e-3


def _dot(a, b):
    return jnp.dot(a, b, preferred_element_type=jnp.float32)


def _dot_rt(a, bt):
    return jax.lax.dot_general(a, bt, (((1,), (1,)), ((), ())),
                               preferred_element_type=jnp.float32)


_PIECES = (
    ((0, 0, 256), (1, 0, 256), (2, 0, 256), (3, 0, 128)),
    ((3, 128, 256), (4, 0, 256), (5, 0, 256), (6, 0, 256)),
)


def _head_kernel(a_ref, b_ref, wa_ref, wb_ref, g1_ref, be1_ref,
                 w2_ref, g2_ref, be2_ref,
                 lw_ref, lb_ref, dw_ref, db_ref,
                 logits_ref, probs_ref, deltas_ref, acc_ref, x2_ref):
    k = pl.program_id(0)
    wa = wa_ref[0]
    wb = wb_ref[0]

    def partial(x3):
        ps = None
        for half, pieces in zip((wa, wb), _PIECES):
            r = 0
            for wi, c0, c1 in pieces:
                d = _dot(x3[wi][:, c0:c1], half[r:r + (c1 - c0), :])
                r += c1 - c0
                ps = d if ps is None else ps + d
        return ps

    pa = partial(a_ref[0])
    pb = partial(b_ref[0])

    @pl.when(k == 0)
    def _():
        acc_ref[0:_NA, :] = pa
        acc_ref[_NA:_N, :] = pb

    @pl.when((k > 0) & (k < 6))
    def _():
        acc_ref[0:_NA, :] += pa
        acc_ref[_NA:_N, :] += pb

    @pl.when(k == 6)
    def _():
        zero = jnp.zeros((1, _H), jnp.float32)

        ta = acc_ref[0:_NA, :] + pa
        tb = acc_ref[_NA:_N, :] + pb
        acc_ref[0:_NA, :] = ta
        acc_ref[_NA:_N, :] = tb
        s1 = jnp.sum(ta, axis=0, keepdims=True) + jnp.sum(tb, axis=0, keepdims=True)
        ss1 = (jnp.sum(ta * ta, axis=0, keepdims=True) +
               jnp.sum(tb * tb, axis=0, keepdims=True))
        mean1 = s1 * (1.0 / _N)
        var1 = ss1 * (1.0 / _N) - mean1 * mean1
        scale1 = g1_ref[...] * jax.lax.rsqrt(var1 + _EPS)
        shift1 = be1_ref[...] - mean1 * scale1

        def body1(i, carry):
            s, ss = carry
            r = i * _RT
            xt = jnp.maximum(acc_ref[pl.ds(r, _RT), :] * scale1 + shift1, 0.0)
            x2t = _dot(xt, w2_ref[...])
            x2_ref[pl.ds(r, _RT), :] = x2t
            return (s + jnp.sum(x2t, axis=0, keepdims=True),
                    ss + jnp.sum(x2t * x2t, axis=0, keepdims=True))

        s2, ss2 = jax.lax.fori_loop(0, _N // _RT, body1, (zero, zero))
        mean2 = s2 * (1.0 / _N)
        var2 = ss2 * (1.0 / _N) - mean2 * mean2
        scale2 = g2_ref[...] * jax.lax.rsqrt(var2 + _EPS)
        shift2 = be2_ref[...] - mean2 * scale2

        def body2(i, carry):
            r = i * _RT
            xt = jnp.maximum(x2_ref[pl.ds(r, _RT), :] * scale2 + shift2, 0.0)
            lg = _dot_rt(lw_ref[...], xt) + lb_ref[...]
            logits_ref[:, pl.ds(r, _RT)] = lg
            m = jnp.max(lg, axis=0, keepdims=True)
            e = jnp.exp(lg - m)
            probs_ref[:, pl.ds(r, _RT)] = e / jnp.sum(e, axis=0, keepdims=True)
            deltas_ref[:, pl.ds(r, _RT)] = _dot_rt(dw_ref[...], xt) + db_ref[...]
            return carry

        jax.lax.fori_loop(0, _N // _RT, body2, 0)


def kernel(pooled_rois_a, pooled_rois_b, conv1_w, conv1_b, bn1_gamma, bn1_beta,
           conv2_w, conv2_b, bn2_gamma, bn2_beta, logits_w, logits_b,
           delta_w, delta_b):
    del conv1_b, conv2_b
    a4 = jnp.transpose(pooled_rois_a, (1, 2, 0, 3))
    b4 = jnp.transpose(pooled_rois_b, (1, 2, 0, 3))
    w3 = conv1_w.reshape(14, _K // 14, _H)
    lwt = logits_w.T
    dwt = delta_w.T
    row = lambda v: v.reshape(1, -1)
    col = lambda v: v.reshape(-1, 1)
    full = lambda shape: pl.BlockSpec(shape, lambda k: tuple(0 for _ in shape))

    logits_t, probs_t, deltas_t = pl.pallas_call(
        _head_kernel,
        grid=(7,),
        in_specs=[
            pl.BlockSpec((1, 7, _NA, 256), lambda k: (k, 0, 0, 0)),
            pl.BlockSpec((1, 7, _NB, 256), lambda k: (k, 0, 0, 0)),
            pl.BlockSpec((1, _K // 14, _H), lambda k: (2 * k, 0, 0)),
            pl.BlockSpec((1, _K // 14, _H), lambda k: (2 * k + 1, 0, 0)),
            full((1, _H)), full((1, _H)),
            full((_H, _H)), full((1, _H)), full((1, _H)),
            full((_NUM_CLASSES, _H)), full((_NUM_CLASSES, 1)),
            full((4 * _NUM_CLASSES, _H)), full((4 * _NUM_CLASSES, 1)),
        ],
        out_specs=[
            full((_NUM_CLASSES, _N)),
            full((_NUM_CLASSES, _N)),
            full((4 * _NUM_CLASSES, _N)),
        ],
        out_shape=[
            jax.ShapeDtypeStruct((_NUM_CLASSES, _N), jnp.float32),
            jax.ShapeDtypeStruct((_NUM_CLASSES, _N), jnp.float32),
            jax.ShapeDtypeStruct((4 * _NUM_CLASSES, _N), jnp.float32),
        ],
        scratch_shapes=[
            pltpu.VMEM((_N, _H), jnp.float32),
            pltpu.VMEM((_N, _H), jnp.float32),
        ],
        compiler_params=pltpu.CompilerParams(
            dimension_semantics=("arbitrary",)),
    )(a4, b4, w3, w3,
      row(bn1_gamma), row(bn1_beta),
      conv2_w, row(bn2_gamma), row(bn2_beta),
      lwt, col(logits_b), dwt, col(delta_b))

    logits = logits_t.T
    probs = probs_t.T
    deltas = deltas_t.reshape(_NUM_CLASSES, 4, _N).transpose(2, 0, 1)
    return (logits, probs, deltas)


# speedup vs baseline: 1.1870x; 1.0391x over previous
import jax
import jax.numpy as jnp
from jax.experimental import pallas as pl
from jax.experimental.pallas import tpu as pltpu

_NUM_CLASSES = 81
_K = 12544
_NA = 512
_NB = 512
_N = _NA + _NB
_H = 1024
_RT = 256
_EPS = 1e-3


def _dot(a, b):
    return jnp.dot(a, b, preferred_element_type=jnp.float32)


def _dot_rt(a, bt):
    return jax.lax.dot_general(a, bt, (((1,), (1,)), ((), ())),
                               preferred_element_type=jnp.float32)


def _head_kernel(a_ref, b_ref, w1_ref, g1_ref, be1_ref,
                 w2_ref, g2_ref, be2_ref,
                 lw_ref, lb_ref, dw_ref, db_ref,
                 logits_ref, probs_ref, deltas_ref, acc_ref, x2_ref):
    k = pl.program_id(0)
    w = w1_ref[...]

    def partial(x3):
        ps = None
        for wi in range(7):
            d = _dot(x3[wi], w[wi * 256:(wi + 1) * 256, :])
            ps = d if ps is None else ps + d
        return ps

    pa = partial(a_ref[0])
    pb = partial(b_ref[0])

    @pl.when(k == 0)
    def _():
        acc_ref[0:_NA, :] = pa
        acc_ref[_NA:_N, :] = pb

    @pl.when((k > 0) & (k < 6))
    def _():
        acc_ref[0:_NA, :] += pa
        acc_ref[_NA:_N, :] += pb

    @pl.when(k == 6)
    def _():
        zero = jnp.zeros((1, _H), jnp.float32)

        ta = acc_ref[0:_NA, :] + pa
        tb = acc_ref[_NA:_N, :] + pb
        acc_ref[0:_NA, :] = ta
        acc_ref[_NA:_N, :] = tb
        s1 = jnp.sum(ta, axis=0, keepdims=True) + jnp.sum(tb, axis=0, keepdims=True)
        ss1 = (jnp.sum(ta * ta, axis=0, keepdims=True) +
               jnp.sum(tb * tb, axis=0, keepdims=True))
        mean1 = s1 * (1.0 / _N)
        var1 = ss1 * (1.0 / _N) - mean1 * mean1
        scale1 = g1_ref[...] * jax.lax.rsqrt(var1 + _EPS)
        shift1 = be1_ref[...] - mean1 * scale1

        def body1(i, carry):
            s, ss = carry
            r = i * _RT
            xt = jnp.maximum(acc_ref[pl.ds(r, _RT), :] * scale1 + shift1, 0.0)
            x2t = _dot(xt, w2_ref[...])
            x2_ref[pl.ds(r, _RT), :] = x2t
            return (s + jnp.sum(x2t, axis=0, keepdims=True),
                    ss + jnp.sum(x2t * x2t, axis=0, keepdims=True))

        s2, ss2 = jax.lax.fori_loop(0, _N // _RT, body1, (zero, zero))
        mean2 = s2 * (1.0 / _N)
        var2 = ss2 * (1.0 / _N) - mean2 * mean2
        scale2 = g2_ref[...] * jax.lax.rsqrt(var2 + _EPS)
        shift2 = be2_ref[...] - mean2 * scale2

        def body2(i, carry):
            r = i * _RT
            xt = jnp.maximum(x2_ref[pl.ds(r, _RT), :] * scale2 + shift2, 0.0)
            lg = _dot_rt(lw_ref[...], xt) + lb_ref[...]
            logits_ref[:, pl.ds(r, _RT)] = lg
            m = jnp.max(lg, axis=0, keepdims=True)
            e = jnp.exp(lg - m)
            probs_ref[:, pl.ds(r, _RT)] = e / jnp.sum(e, axis=0, keepdims=True)
            deltas_ref[:, pl.ds(r, _RT)] = _dot_rt(dw_ref[...], xt) + db_ref[...]
            return carry

        jax.lax.fori_loop(0, _N // _RT, body2, 0)


def kernel(pooled_rois_a, pooled_rois_b, conv1_w, conv1_b, bn1_gamma, bn1_beta,
           conv2_w, conv2_b, bn2_gamma, bn2_beta, logits_w, logits_b,
           delta_w, delta_b):
    del conv1_b, conv2_b
    a4 = jnp.transpose(pooled_rois_a, (1, 2, 0, 3))
    b4 = jnp.transpose(pooled_rois_b, (1, 2, 0, 3))
    lwt = logits_w.T
    dwt = delta_w.T
    row = lambda v: v.reshape(1, -1)
    col = lambda v: v.reshape(-1, 1)
    full = lambda shape: pl.BlockSpec(shape, lambda k: tuple(0 for _ in shape))

    logits_t, probs_t, deltas_t = pl.pallas_call(
        _head_kernel,
        grid=(7,),
        in_specs=[
            pl.BlockSpec((1, 7, _NA, 256), lambda k: (k, 0, 0, 0)),
            pl.BlockSpec((1, 7, _NB, 256), lambda k: (k, 0, 0, 0)),
            pl.BlockSpec((_K // 7, _H), lambda k: (k, 0)),
            full((1, _H)), full((1, _H)),
            full((_H, _H)), full((1, _H)), full((1, _H)),
            full((_NUM_CLASSES, _H)), full((_NUM_CLASSES, 1)),
            full((4 * _NUM_CLASSES, _H)), full((4 * _NUM_CLASSES, 1)),
        ],
        out_specs=[
            full((_NUM_CLASSES, _N)),
            full((_NUM_CLASSES, _N)),
            full((4 * _NUM_CLASSES, _N)),
        ],
        out_shape=[
            jax.ShapeDtypeStruct((_NUM_CLASSES, _N), jnp.float32),
            jax.ShapeDtypeStruct((_NUM_CLASSES, _N), jnp.float32),
            jax.ShapeDtypeStruct((4 * _NUM_CLASSES, _N), jnp.float32),
        ],
        scratch_shapes=[
            pltpu.VMEM((_N, _H), jnp.float32),
            pltpu.VMEM((_N, _H), jnp.float32),
        ],
        compiler_params=pltpu.CompilerParams(
            dimension_semantics=("arbitrary",)),
    )(a4, b4, conv1_w,
      row(bn1_gamma), row(bn1_beta),
      conv2_w, row(bn2_gamma), row(bn2_beta),
      lwt, col(logits_b), dwt, col(delta_b))

    logits = logits_t.T
    probs = probs_t.T
    deltas = deltas_t.reshape(_NUM_CLASSES, 4, _N).transpose(2, 0, 1)
    return (logits, probs, deltas)


# speedup vs baseline: 1.2126x; 1.0216x over previous
import jax
import jax.numpy as jnp
from jax.experimental import pallas as pl
from jax.experimental.pallas import tpu as pltpu

_NUM_CLASSES = 81
_K = 12544
_NA = 512
_NB = 512
_N = _NA + _NB
_H = 1024
_RT = 512
_EPS = 1e-3


def _dot(a, b):
    return jnp.dot(a, b, preferred_element_type=jnp.float32)


def _dot_rt(a, bt):
    return jax.lax.dot_general(a, bt, (((1,), (1,)), ((), ())),
                               preferred_element_type=jnp.float32)


def _head_kernel(a_ref, b_ref, w1_ref, g1_ref, be1_ref,
                 w2_ref, g2_ref, be2_ref,
                 lw_ref, lb_ref, dw_ref, db_ref,
                 logits_ref, probs_ref, deltas_ref, acc_ref, x2_ref):
    k = pl.program_id(0)
    w = w1_ref[...]

    def partial(x3):
        ps = None
        for wi in range(7):
            d = _dot(x3[wi], w[wi * 256:(wi + 1) * 256, :])
            ps = d if ps is None else ps + d
        return ps

    pa = partial(a_ref[0])
    pb = partial(b_ref[0])

    @pl.when(k == 0)
    def _():
        acc_ref[0:_NA, :] = pa
        acc_ref[_NA:_N, :] = pb

    @pl.when((k > 0) & (k < 6))
    def _():
        acc_ref[0:_NA, :] += pa
        acc_ref[_NA:_N, :] += pb

    @pl.when(k == 6)
    def _():
        zero = jnp.zeros((1, _H), jnp.float32)

        ta = acc_ref[0:_NA, :] + pa
        tb = acc_ref[_NA:_N, :] + pb
        acc_ref[0:_NA, :] = ta
        acc_ref[_NA:_N, :] = tb
        s1 = jnp.sum(ta, axis=0, keepdims=True) + jnp.sum(tb, axis=0, keepdims=True)
        ss1 = (jnp.sum(ta * ta, axis=0, keepdims=True) +
               jnp.sum(tb * tb, axis=0, keepdims=True))
        mean1 = s1 * (1.0 / _N)
        var1 = ss1 * (1.0 / _N) - mean1 * mean1
        scale1 = g1_ref[...] * jax.lax.rsqrt(var1 + _EPS)
        shift1 = be1_ref[...] - mean1 * scale1

        def body1(i, carry):
            s, ss = carry
            r = i * _RT
            xt = jnp.maximum(acc_ref[pl.ds(r, _RT), :] * scale1 + shift1, 0.0)
            x2t = _dot(xt, w2_ref[...])
            x2_ref[pl.ds(r, _RT), :] = x2t
            return (s + jnp.sum(x2t, axis=0, keepdims=True),
                    ss + jnp.sum(x2t * x2t, axis=0, keepdims=True))

        s2, ss2 = jax.lax.fori_loop(0, _N // _RT, body1, (zero, zero))
        mean2 = s2 * (1.0 / _N)
        var2 = ss2 * (1.0 / _N) - mean2 * mean2
        scale2 = g2_ref[...] * jax.lax.rsqrt(var2 + _EPS)
        shift2 = be2_ref[...] - mean2 * scale2

        def body2(i, carry):
            r = i * _RT
            xt = jnp.maximum(x2_ref[pl.ds(r, _RT), :] * scale2 + shift2, 0.0)
            lg = _dot_rt(lw_ref[...], xt) + lb_ref[...]
            logits_ref[:, pl.ds(r, _RT)] = lg
            m = jnp.max(lg, axis=0, keepdims=True)
            e = jnp.exp(lg - m)
            probs_ref[:, pl.ds(r, _RT)] = e / jnp.sum(e, axis=0, keepdims=True)
            deltas_ref[:, pl.ds(r, _RT)] = _dot_rt(dw_ref[...], xt) + db_ref[...]
            return carry

        jax.lax.fori_loop(0, _N // _RT, body2, 0)


def kernel(pooled_rois_a, pooled_rois_b, conv1_w, conv1_b, bn1_gamma, bn1_beta,
           conv2_w, conv2_b, bn2_gamma, bn2_beta, logits_w, logits_b,
           delta_w, delta_b):
    del conv1_b, conv2_b
    a4 = jnp.transpose(pooled_rois_a, (1, 2, 0, 3))
    b4 = jnp.transpose(pooled_rois_b, (1, 2, 0, 3))
    lwt = logits_w.T
    dwt = delta_w.T
    row = lambda v: v.reshape(1, -1)
    col = lambda v: v.reshape(-1, 1)
    full = lambda shape: pl.BlockSpec(shape, lambda k: tuple(0 for _ in shape))

    logits_t, probs_t, deltas_t = pl.pallas_call(
        _head_kernel,
        grid=(7,),
        in_specs=[
            pl.BlockSpec((1, 7, _NA, 256), lambda k: (k, 0, 0, 0)),
            pl.BlockSpec((1, 7, _NB, 256), lambda k: (k, 0, 0, 0)),
            pl.BlockSpec((_K // 7, _H), lambda k: (k, 0)),
            full((1, _H)), full((1, _H)),
            full((_H, _H)), full((1, _H)), full((1, _H)),
            full((_NUM_CLASSES, _H)), full((_NUM_CLASSES, 1)),
            full((4 * _NUM_CLASSES, _H)), full((4 * _NUM_CLASSES, 1)),
        ],
        out_specs=[
            full((_NUM_CLASSES, _N)),
            full((_NUM_CLASSES, _N)),
            full((4 * _NUM_CLASSES, _N)),
        ],
        out_shape=[
            jax.ShapeDtypeStruct((_NUM_CLASSES, _N), jnp.float32),
            jax.ShapeDtypeStruct((_NUM_CLASSES, _N), jnp.float32),
            jax.ShapeDtypeStruct((4 * _NUM_CLASSES, _N), jnp.float32),
        ],
        scratch_shapes=[
            pltpu.VMEM((_N, _H), jnp.float32),
            pltpu.VMEM((_N, _H), jnp.float32),
        ],
        compiler_params=pltpu.CompilerParams(
            dimension_semantics=("arbitrary",)),
    )(a4, b4, conv1_w,
      row(bn1_gamma), row(bn1_beta),
      conv2_w, row(bn2_gamma), row(bn2_beta),
      lwt, col(logits_b), dwt, col(delta_b))

    logits = logits_t.T
    probs = probs_t.T
    deltas = deltas_t.reshape(_NUM_CLASSES, 4, _N).transpose(2, 0, 1)
    return (logits, probs, deltas)
